# Initial kernel scaffold; baseline (speedup 1.0000x reference)
import functools

import jax
import jax.numpy as jnp
from jax import lax
from jax.experimental import pallas as pl
from jax.experimental.pallas import tpu as pltpu
from jax.experimental.pallas import tpu_sc as plsc

N = 10000
E = 320000
D = 128
EPS = 1e-5

NC = 2
NS = 16
NW = NC * NS
CHUNK = 128
CHUNKS_PER_W = 80
E_PAD = NW * CHUNKS_PER_W * CHUNK
ROWS_PER_TILE = N // NS
STAGE = 125


def _sc_conv_body(x_hbm, srcg, dstg, ewg, out_hbm,
                  src_v, dst_v, ew_v, rows_v, stage_v, acc):
    cid = lax.axis_index("c")
    sid = lax.axis_index("s")
    wid = cid * NS + sid

    def zero_row(i, carry):
        for c in range(D // 16):
            stage_v[i, pl.ds(c * 16, 16)] = jnp.zeros((16,), jnp.float32)
        return carry
    lax.fori_loop(0, STAGE, zero_row, 0)
    for t in range(ROWS_PER_TILE // STAGE):
        pltpu.sync_copy(stage_v, acc.at[pl.ds(sid * ROWS_PER_TILE + t * STAGE, STAGE)])
    plsc.subcore_barrier()

    pltpu.sync_copy(srcg.at[wid], src_v)
    pltpu.sync_copy(dstg.at[wid], dst_v)
    pltpu.sync_copy(ewg.at[wid], ew_v)

    def chunk(j, carry):
        pltpu.sync_copy(x_hbm.at[src_v.at[j]], rows_v)

        def edge(i, c2):
            w = ew_v[j, i]
            for c in range(D // 16):
                sl = pl.ds(c * 16, 16)
                rows_v[i, sl] = rows_v[i, sl] * w
            return c2
        lax.fori_loop(0, CHUNK, edge, 0)
        pltpu.sync_copy(rows_v, acc.at[dst_v.at[j]], add=True)
        return carry
    lax.fori_loop(0, CHUNKS_PER_W, chunk, 0)
    plsc.subcore_barrier()

    for t in range(ROWS_PER_TILE // STAGE):
        r0 = sid * ROWS_PER_TILE + t * STAGE
        pltpu.sync_copy(acc.at[pl.ds(r0, STAGE)], stage_v)
        pltpu.sync_copy(stage_v, out_hbm.at[cid, pl.ds(r0, STAGE)])


_sc_conv = pl.kernel(
    _sc_conv_body,
    out_type=jax.ShapeDtypeStruct((NC, N, D), jnp.float32),
    mesh=plsc.VectorSubcoreMesh(core_axis_name="c", subcore_axis_name="s"),
    scratch_types=[
        pltpu.VMEM((CHUNKS_PER_W, CHUNK), jnp.int32),
        pltpu.VMEM((CHUNKS_PER_W, CHUNK), jnp.int32),
        pltpu.VMEM((CHUNKS_PER_W, CHUNK), jnp.float32),
        pltpu.VMEM((CHUNK, D), jnp.float32),
        pltpu.VMEM((STAGE, D), jnp.float32),
        pltpu.VMEM_SHARED((N, D), jnp.float32),
    ],
)


def _tc_post_body(p_ref, x_ref, wrel_ref, wroot_ref, b_ref, g_ref, be_ref,
                  o_ref, *, sig):
    agg = p_ref[0] + p_ref[1]
    h = lax.dot_general(agg, wrel_ref[...], (((1,), (1,)), ((), ())),
                        preferred_element_type=jnp.float32)
    h = h + lax.dot_general(x_ref[...], wroot_ref[...], (((1,), (1,)), ((), ())),
                            preferred_element_type=jnp.float32)
    h = h + b_ref[...]
    h = jnp.maximum(h, 0.0)
    mu = jnp.mean(h, axis=0, keepdims=True)
    var = jnp.mean((h - mu) * (h - mu), axis=0, keepdims=True)
    y = (h - mu) * lax.rsqrt(var + EPS) * g_ref[...] + be_ref[...]
    if sig:
        y = jax.nn.sigmoid(y)
    o_ref[...] = y


def _tc_post(p, x, wrel, wroot, b, gamma, beta, sig):
    return pl.pallas_call(
        functools.partial(_tc_post_body, sig=sig),
        out_shape=jax.ShapeDtypeStruct((N, D), jnp.float32),
    )(p, x, wrel, wroot, b.reshape(1, D), gamma.reshape(1, D),
      beta.reshape(1, D))


def kernel(x, edge_index, edge_attr, W_rel0, W_root0, b0, gamma0, beta0,
           W_rel1, W_root1, b1, gamma1, beta1):
    pad = E_PAD - E
    src = jnp.pad(edge_index[0], (0, pad)).reshape(NW, CHUNKS_PER_W, CHUNK)
    dst = jnp.pad(edge_index[1], (0, pad)).reshape(NW, CHUNKS_PER_W, CHUNK)
    ew = jnp.pad(edge_attr, (0, pad)).reshape(NW, CHUNKS_PER_W, CHUNK)

    p0 = _sc_conv(x, src, dst, ew)
    h1 = _tc_post(p0, x, W_rel0, W_root0, b0, gamma0, beta0, False)
    p1 = _sc_conv(h1, src, dst, ew)
    return _tc_post(p1, h1, W_rel1, W_root1, b1, gamma1, beta1, True)


# speedup vs baseline: 3.2862x; 3.2862x over previous
import functools

import jax
import jax.numpy as jnp
from jax import lax
from jax.experimental import pallas as pl
from jax.experimental.pallas import tpu as pltpu
from jax.experimental.pallas import tpu_sc as plsc

N = 10000
E = 320000
D = 128
EPS = 1e-5

NC = 2
NS = 16
NW = NC * NS
CHUNK = 128
CHUNKS_PER_W = 80
E_PAD = NW * CHUNKS_PER_W * CHUNK
N_PAD = 10240
ROWS_PER_TILE = N_PAD // NS
STAGE = 128


def _sc_conv_body(x_hbm, srcg, dstg, ewg, out_hbm,
                  src_v, dst_v, ew_v, rows_v, acc):
    cid = lax.axis_index("c")
    sid = lax.axis_index("s")
    wid = cid * NS + sid

    def zero_row(i, carry):
        for c in range(D // 16):
            rows_v[i, pl.ds(c * 16, 16)] = jnp.zeros((16,), jnp.float32)
        return carry
    lax.fori_loop(0, STAGE, zero_row, 0)
    for t in range(ROWS_PER_TILE // STAGE):
        pltpu.sync_copy(rows_v, acc.at[pl.ds(sid * ROWS_PER_TILE + t * STAGE, STAGE)])
    plsc.subcore_barrier()

    pltpu.sync_copy(srcg.at[wid], src_v)
    pltpu.sync_copy(dstg.at[wid], dst_v)
    pltpu.sync_copy(ewg.at[wid], ew_v)

    def chunk(j, carry):
        pltpu.sync_copy(x_hbm.at[src_v.at[j]], rows_v)

        def group(g, c2):
            wv = ew_v[j, pl.ds(g * 16, 16)]
            for l in range(16):
                w = wv[l]
                i = g * 16 + l
                for c in range(D // 16):
                    sl = pl.ds(c * 16, 16)
                    rows_v[i, sl] = rows_v[i, sl] * w
            return c2
        lax.fori_loop(0, CHUNK // 16, group, 0)
        pltpu.sync_copy(rows_v, acc.at[dst_v.at[j]], add=True)
        return carry
    lax.fori_loop(0, CHUNKS_PER_W, chunk, 0)
    plsc.subcore_barrier()

    for t in range(ROWS_PER_TILE // STAGE):
        r0 = sid * ROWS_PER_TILE + t * STAGE
        pltpu.sync_copy(acc.at[pl.ds(r0, STAGE)], rows_v)
        pltpu.sync_copy(rows_v, out_hbm.at[cid, pl.ds(r0, STAGE)])


@functools.cache
def _sc_conv_kernel():
    return pl.kernel(
        _sc_conv_body,
        out_type=jax.ShapeDtypeStruct((NC, N_PAD, D), jnp.float32),
        mesh=plsc.VectorSubcoreMesh(core_axis_name="c", subcore_axis_name="s",
                                    num_cores=NC, num_subcores=NS),
        scratch_types=[
            pltpu.VMEM((CHUNKS_PER_W, CHUNK), jnp.int32),
            pltpu.VMEM((CHUNKS_PER_W, CHUNK), jnp.int32),
            pltpu.VMEM((CHUNKS_PER_W, CHUNK), jnp.float32),
            pltpu.VMEM((CHUNK, D), jnp.float32),
            pltpu.VMEM_SHARED((N_PAD, D), jnp.float32),
        ],
    )


def _sc_conv(x, src, dst, ew):
    return _sc_conv_kernel()(x, src, dst, ew)


def _tc_post_body(p_ref, x_ref, wrel_ref, wroot_ref, b_ref, g_ref, be_ref,
                  o_ref, *, sig):
    agg = p_ref[0, :N, :] + p_ref[1, :N, :]
    h = lax.dot_general(agg, wrel_ref[...], (((1,), (1,)), ((), ())),
                        preferred_element_type=jnp.float32)
    h = h + lax.dot_general(x_ref[...], wroot_ref[...], (((1,), (1,)), ((), ())),
                            preferred_element_type=jnp.float32)
    h = h + b_ref[...]
    h = jnp.maximum(h, 0.0)
    mu = jnp.mean(h, axis=0, keepdims=True)
    var = jnp.mean((h - mu) * (h - mu), axis=0, keepdims=True)
    y = (h - mu) * lax.rsqrt(var + EPS) * g_ref[...] + be_ref[...]
    if sig:
        y = jax.nn.sigmoid(y)
    o_ref[...] = y


def _tc_post(p, x, wrel, wroot, b, gamma, beta, sig):
    return pl.pallas_call(
        functools.partial(_tc_post_body, sig=sig),
        out_shape=jax.ShapeDtypeStruct((N, D), jnp.float32),
    )(p, x, wrel, wroot, b.reshape(1, D), gamma.reshape(1, D),
      beta.reshape(1, D))


def kernel(x, edge_index, edge_attr, W_rel0, W_root0, b0, gamma0, beta0,
           W_rel1, W_root1, b1, gamma1, beta1):
    pad = E_PAD - E
    src = jnp.pad(edge_index[0], (0, pad)).reshape(NW, CHUNKS_PER_W, CHUNK)
    dst = jnp.pad(edge_index[1], (0, pad)).reshape(NW, CHUNKS_PER_W, CHUNK)
    ew = jnp.pad(edge_attr, (0, pad)).reshape(NW, CHUNKS_PER_W, CHUNK)

    p0 = _sc_conv(x, src, dst, ew)
    h1 = _tc_post(p0, x, W_rel0, W_root0, b0, gamma0, beta0, False)
    p1 = _sc_conv(h1, src, dst, ew)
    return _tc_post(p1, h1, W_rel1, W_root1, b1, gamma1, beta1, True)
